# baseline (device time: 15975 ns/iter reference)
import jax
import jax.numpy as jnp
from jax import lax
from jax.experimental import pallas as pl
from jax.experimental.pallas import tpu as pltpu

N_DEV = 4

_SEND_ORDER = (1, 0, 2)
_RECV_ORDER = (0, 2, 1)


def kernel(A, B):
    m, k = A.shape
    _, n = B.shape
    ch = m // N_DEV

    def body(a_ref, b_ref, out_ref, send_chunks, rs_buf, red_buf, ag_src,
             ag_buf, rs_send_sems, rs_sems, ag_send_sems, ag_sems):
        my = lax.axis_index("i")

        barrier_sem = pltpu.get_barrier_semaphore()
        for h in range(N_DEV - 1):
            pl.semaphore_signal(
                barrier_sem, inc=1,
                device_id=((my + 1 + h) % N_DEV,),
                device_id_type=pl.DeviceIdType.MESH,
            )

        b_bf16 = b_ref[...].astype(jnp.bfloat16)

        rdmas = []
        for i, h in enumerate(_SEND_ORDER):
            dest = (my + 1 + h) % N_DEV
            chunk = jnp.dot(
                a_ref[pl.ds(dest * ch, ch), :].astype(jnp.bfloat16),
                b_bf16,
                preferred_element_type=jnp.float32,
            )
            send_chunks[h] = chunk.astype(jnp.bfloat16)
            if i == 0:
                pl.semaphore_wait(barrier_sem, N_DEV - 1)
            rdma = pltpu.make_async_remote_copy(
                src_ref=send_chunks.at[h],
                dst_ref=rs_buf.at[2 - h],
                send_sem=rs_send_sems.at[h],
                recv_sem=rs_sems.at[2 - h],
                device_id=(dest,),
                device_id_type=pl.DeviceIdType.MESH,
            )
            rdma.start()
            rdmas.append(rdma)

        red_buf[...] = jnp.dot(
            a_ref[pl.ds(my * ch, ch), :].astype(jnp.bfloat16),
            b_bf16,
            preferred_element_type=jnp.float32,
        )

        for j in _RECV_ORDER:
            recv = pltpu.make_async_remote_copy(
                src_ref=rs_buf.at[j],
                dst_ref=rs_buf.at[j],
                send_sem=rs_send_sems.at[0],
                recv_sem=rs_sems.at[j],
                device_id=(my,),
                device_id_type=pl.DeviceIdType.MESH,
            )
            recv.wait_recv()
            red_buf[...] = red_buf[...] + rs_buf[j].astype(jnp.float32)

        red = jnp.maximum(red_buf[...], 0.0)
        ag_src[...] = red.astype(jnp.bfloat16)

        for h in _SEND_ORDER:
            dest = (my + 1 + h) % N_DEV
            rdma = pltpu.make_async_remote_copy(
                src_ref=ag_src,
                dst_ref=ag_buf.at[2 - h],
                send_sem=ag_send_sems.at[h],
                recv_sem=ag_sems.at[2 - h],
                device_id=(dest,),
                device_id_type=pl.DeviceIdType.MESH,
            )
            rdma.start()
            rdmas.append(rdma)

        out_ref[pl.ds(my * ch, ch), :] = red
        for j in _RECV_ORDER:
            recv = pltpu.make_async_remote_copy(
                src_ref=ag_buf.at[j],
                dst_ref=ag_buf.at[j],
                send_sem=ag_send_sems.at[0],
                recv_sem=ag_sems.at[j],
                device_id=(my,),
                device_id_type=pl.DeviceIdType.MESH,
            )
            recv.wait_recv()
            src_chip = (my + 1 + j) % N_DEV
            out_ref[pl.ds(src_chip * ch, ch), :] = (
                ag_buf[j].astype(jnp.float32)
            )

        for rdma in rdmas:
            rdma.wait_send()

    return pl.pallas_call(
        body,
        out_shape=jax.ShapeDtypeStruct((m, n), jnp.float32),
        in_specs=[
            pl.BlockSpec(memory_space=pltpu.VMEM),
            pl.BlockSpec(memory_space=pltpu.VMEM),
        ],
        out_specs=pl.BlockSpec(memory_space=pltpu.VMEM),
        scratch_shapes=[
            pltpu.VMEM((N_DEV - 1, ch, n), jnp.bfloat16),
            pltpu.VMEM((N_DEV - 1, ch, n), jnp.bfloat16),
            pltpu.VMEM((ch, n), jnp.float32),
            pltpu.VMEM((ch, n), jnp.bfloat16),
            pltpu.VMEM((N_DEV - 1, ch, n), jnp.bfloat16),
            pltpu.SemaphoreType.DMA((N_DEV - 1,)),
            pltpu.SemaphoreType.DMA((N_DEV - 1,)),
            pltpu.SemaphoreType.DMA((N_DEV - 1,)),
            pltpu.SemaphoreType.DMA((N_DEV - 1,)),
        ],
        compiler_params=pltpu.CompilerParams(collective_id=0),
    )(A, B)


# device time: 14172 ns/iter; 1.1272x vs baseline; 1.1272x over previous
import jax
import jax.numpy as jnp
from jax import lax
from jax.experimental import pallas as pl
from jax.experimental.pallas import tpu as pltpu

N_DEV = 4
SUBS = 4

_SEND_ORDER = (1, 0, 2)
_RECV_ORDER = (0, 2, 1)


def kernel(A, B):
    m, k = A.shape
    _, n = B.shape
    ch = m // N_DEV
    sub = ch // SUBS

    def body(a_ref, b_ref, out_ref, send_buf, rs_buf, ag_src,
             rs_send_sems, rs_sems, ag_send_sems, ag_sems):
        my = lax.axis_index("i")

        barrier_sem = pltpu.get_barrier_semaphore()
        for h in range(N_DEV - 1):
            pl.semaphore_signal(
                barrier_sem, inc=1,
                device_id=((my + 1 + h) % N_DEV,),
                device_id_type=pl.DeviceIdType.MESH,
            )

        partial = jnp.dot(
            a_ref[...].astype(jnp.bfloat16),
            b_ref[...].astype(jnp.bfloat16),
            preferred_element_type=jnp.float32,
        )
        send_buf[...] = partial.astype(jnp.bfloat16)

        pl.semaphore_wait(barrier_sem, N_DEV - 1)

        rdmas = []
        for s in range(SUBS):
            for h in _SEND_ORDER:
                dest = (my + 1 + h) % N_DEV
                rdma = pltpu.make_async_remote_copy(
                    src_ref=send_buf.at[pl.ds(dest * ch + s * sub, sub), :],
                    dst_ref=rs_buf.at[2 - h, s],
                    send_sem=rs_send_sems.at[h, s],
                    recv_sem=rs_sems.at[2 - h, s],
                    device_id=(dest,),
                    device_id_type=pl.DeviceIdType.MESH,
                )
                rdma.start()
                rdmas.append(rdma)

        for s in range(SUBS):
            red = send_buf[
                pl.ds(my * ch + s * sub, sub), :
            ].astype(jnp.float32)
            for j in _RECV_ORDER:
                recv = pltpu.make_async_remote_copy(
                    src_ref=rs_buf.at[j, s],
                    dst_ref=rs_buf.at[j, s],
                    send_sem=rs_send_sems.at[0, s],
                    recv_sem=rs_sems.at[j, s],
                    device_id=(my,),
                    device_id_type=pl.DeviceIdType.MESH,
                )
                recv.wait_recv()
                red = red + rs_buf[j, s].astype(jnp.float32)
            red_bf = jnp.maximum(red, 0.0).astype(jnp.bfloat16)
            ag_src[s] = red_bf
            for h in _SEND_ORDER:
                dest = (my + 1 + h) % N_DEV
                rdma = pltpu.make_async_remote_copy(
                    src_ref=ag_src.at[s],
                    dst_ref=out_ref.at[pl.ds(my * ch + s * sub, sub), :],
                    send_sem=ag_send_sems.at[h, s],
                    recv_sem=ag_sems.at[2 - h, s],
                    device_id=(dest,),
                    device_id_type=pl.DeviceIdType.MESH,
                )
                rdma.start()
                rdmas.append(rdma)
            out_ref[pl.ds(my * ch + s * sub, sub), :] = red_bf

        for s in range(SUBS):
            for j in _RECV_ORDER:
                src_chip = (my + 1 + j) % N_DEV
                recv = pltpu.make_async_remote_copy(
                    src_ref=ag_src.at[s],
                    dst_ref=out_ref.at[
                        pl.ds(src_chip * ch + s * sub, sub), :
                    ],
                    send_sem=ag_send_sems.at[0, s],
                    recv_sem=ag_sems.at[j, s],
                    device_id=(my,),
                    device_id_type=pl.DeviceIdType.MESH,
                )
                recv.wait_recv()

        for rdma in rdmas:
            rdma.wait_send()

    return pl.pallas_call(
        body,
        out_shape=jax.ShapeDtypeStruct((m, n), jnp.bfloat16),
        in_specs=[
            pl.BlockSpec(memory_space=pltpu.VMEM),
            pl.BlockSpec(memory_space=pltpu.VMEM),
        ],
        out_specs=pl.BlockSpec(memory_space=pltpu.VMEM),
        scratch_shapes=[
            pltpu.VMEM((m, n), jnp.bfloat16),
            pltpu.VMEM((N_DEV - 1, SUBS, sub, n), jnp.bfloat16),
            pltpu.VMEM((SUBS, sub, n), jnp.bfloat16),
            pltpu.SemaphoreType.DMA((N_DEV - 1, SUBS)),
            pltpu.SemaphoreType.DMA((N_DEV - 1, SUBS)),
            pltpu.SemaphoreType.DMA((N_DEV - 1, SUBS)),
            pltpu.SemaphoreType.DMA((N_DEV - 1, SUBS)),
        ],
        compiler_params=pltpu.CompilerParams(collective_id=0),
    )(A, B)
